# baseline (speedup 1.0000x reference)
import jax
import jax.numpy as jnp
from jax.experimental import pallas as pl
from jax.experimental.pallas import tpu as pltpu

_TB = 256


def _prologue(e_bt, wt_bt, b_bt, e_st, wt_st, b_st, e_cj, wt_cj, b_cj,
              wf0, bf0, o_ref):
    f0 = wf0[...]
    u_bt = ((e_bt[...] @ wt_bt[...]) + b_bt[...]) @ f0 + bf0[...]
    u_st = ((e_st[...] @ wt_st[...]) + b_st[...]) @ f0
    u_cj = ((e_cj[...] @ wt_cj[...]) + b_cj[...]) @ f0
    d = f0.shape[0] - 1
    sd_row = f0[d:d + 1, :]
    pad = jnp.zeros((o_ref.shape[0] - 17, f0.shape[1]), jnp.float32)
    o_ref[...] = jnp.concatenate([u_bt, u_st, u_cj, sd_row, pad], axis=0)


def _main(d_ref, sd_ref, bt_ref, st_ref, cj_ref,
          w0_ref, ws_ref, w1_ref, b1_ref, w2_ref, b2_ref, o_ref):
    x = d_ref[...]
    sd = sd_ref[...]
    bt = bt_ref[...]
    st = st_ref[...]
    cj = cj_ref[...]
    nsub = sd.shape[0]
    nc = ws_ref.shape[0]
    iic = jax.lax.broadcasted_iota(jnp.int32, (nc, sd.shape[1]), 0)
    chunks = []
    for s in range(nsub):
        ohc = ((iic == jnp.broadcast_to(bt[s:s + 1, :], iic.shape))
               | (iic == jnp.broadcast_to(st[s:s + 1, :], iic.shape) + 8)
               | (iic == jnp.broadcast_to(cj[s:s + 1, :], iic.shape) + 14))
        sdb = jnp.broadcast_to(sd[s:s + 1, :], iic.shape)
        chunks.append(jnp.where(iic == 16, sdb, ohc.astype(jnp.float32)))
    gt = jnp.concatenate(chunks, axis=1)
    g = jnp.transpose(gt).astype(jnp.bfloat16)
    h0 = jnp.dot(x.astype(jnp.bfloat16), w0_ref[...].astype(jnp.bfloat16),
                 preferred_element_type=jnp.float32).astype(jnp.bfloat16)
    hg = jnp.dot(g, ws_ref[...].astype(jnp.bfloat16),
                 preferred_element_type=jnp.float32).astype(jnp.bfloat16)
    h = jnp.maximum(h0 + hg, jnp.bfloat16(0.0))
    h = jnp.dot(h, w1_ref[...].astype(jnp.bfloat16),
                preferred_element_type=jnp.float32).astype(jnp.bfloat16)
    h = jnp.maximum(h + b1_ref[...].astype(jnp.bfloat16), jnp.bfloat16(0.0))
    o = jnp.sum(h.astype(jnp.float32) * w2_ref[...], axis=1,
                keepdims=True) + b2_ref[...]
    o_ref[...] = o


def kernel(atom, degree, hybridization, chirality, formal_charge, partial_charge, pos, dist, spatial_distance, bond_type, stereo, conjugated, emb_atom, emb_degree, emb_hybridization, emb_chirality, emb_bond_type, emb_stereo, emb_conjugated, W_charge, b_charge, W_pos, b_pos, Wt_bond_type, bt_bond_type, Wt_stereo, bt_stereo, Wt_conjugated, bt_conjugated, Wf0, bf0, Wf1, bf1, Wf2, bf2):
    b, n = spatial_distance.shape[0], spatial_distance.shape[1]
    d = dist.shape[-1]
    h1 = Wf0.shape[1]
    h2 = Wf1.shape[1]
    r = b * n

    wsmall = pl.pallas_call(
        _prologue,
        out_shape=jax.ShapeDtypeStruct((24, h1), jnp.float32),
    )(emb_bond_type, Wt_bond_type, bt_bond_type.reshape(1, -1),
      emb_stereo, Wt_stereo, bt_stereo.reshape(1, -1),
      emb_conjugated, Wt_conjugated, bt_conjugated.reshape(1, -1),
      Wf0, bf0.reshape(1, -1))

    m = r * n
    tm = min(_TB * n, m)
    lanes = 128
    msub = m // lanes
    nsub = tm // lanes
    dist2 = dist.reshape(m, d)
    sd2 = spatial_distance.reshape(msub, lanes)
    bt2 = bond_type.reshape(msub, lanes)
    st2 = stereo.reshape(msub, lanes)
    cj2 = conjugated.reshape(msub, lanes)

    grid = (m // tm,)
    lane_spec = pl.BlockSpec((nsub, lanes), lambda i: (i, 0))
    col_spec = pl.BlockSpec((tm, 1), lambda i: (i, 0))
    out2 = pl.pallas_call(
        _main,
        grid=grid,
        in_specs=[
            pl.BlockSpec((tm, d), lambda i: (i, 0)),
            lane_spec, lane_spec, lane_spec, lane_spec,
            pl.BlockSpec((d, h1), lambda i: (0, 0)),
            pl.BlockSpec((24, h1), lambda i: (0, 0)),
            pl.BlockSpec((h1, h2), lambda i: (0, 0)),
            pl.BlockSpec((1, h2), lambda i: (0, 0)),
            pl.BlockSpec((1, h2), lambda i: (0, 0)),
            pl.BlockSpec((1, 1), lambda i: (0, 0)),
        ],
        out_specs=col_spec,
        out_shape=jax.ShapeDtypeStruct((m, 1), jnp.float32),
        compiler_params=pltpu.CompilerParams(
            dimension_semantics=("parallel",)),
    )(dist2, sd2, bt2, st2, cj2,
      Wf0[:d], wsmall, Wf1, bf1.reshape(1, -1), Wf2.reshape(1, -1),
      bf2.reshape(1, -1))

    return out2.reshape(b, n, n)


# speedup vs baseline: 1.1955x; 1.1955x over previous
import jax
import jax.numpy as jnp
from jax.experimental import pallas as pl
from jax.experimental.pallas import tpu as pltpu

_TB = 256


def _prologue(e_bt, wt_bt, b_bt, e_st, wt_st, b_st, e_cj, wt_cj, b_cj,
              wf0, bf0, o_ref):
    f0 = wf0[...]
    u_bt = ((e_bt[...] @ wt_bt[...]) + b_bt[...]) @ f0 + bf0[...]
    u_st = ((e_st[...] @ wt_st[...]) + b_st[...]) @ f0
    u_cj = ((e_cj[...] @ wt_cj[...]) + b_cj[...]) @ f0
    d = f0.shape[0] - 1
    sd_row = f0[d:d + 1, :]
    pad = jnp.zeros((o_ref.shape[0] - 17, f0.shape[1]), jnp.float32)
    o_ref[...] = jnp.concatenate([u_bt, u_st, u_cj, sd_row, pad], axis=0)


def _main(d_ref, sd_ref, bt_ref, st_ref, cj_ref,
          w0_ref, ws_ref, w1_ref, b1_ref, w2_ref, b2_ref, o_ref):
    x = d_ref[...]
    sd = sd_ref[...]
    bt = bt_ref[...]
    st = st_ref[...]
    cj = cj_ref[...]
    nsub = sd.shape[0]
    nc = ws_ref.shape[0]
    iic = jax.lax.broadcasted_iota(jnp.int32, (nc, sd.shape[1]), 0)
    chunks = []
    for s in range(nsub):
        ohc = ((iic == jnp.broadcast_to(bt[s:s + 1, :], iic.shape))
               | (iic == jnp.broadcast_to(st[s:s + 1, :], iic.shape) + 8)
               | (iic == jnp.broadcast_to(cj[s:s + 1, :], iic.shape) + 14))
        sdb = jnp.broadcast_to(sd[s:s + 1, :], iic.shape)
        chunks.append(jnp.where(iic == 16, sdb, ohc.astype(jnp.float32)))
    gt = jnp.concatenate(chunks, axis=1)
    g = jnp.transpose(gt).astype(jnp.bfloat16)
    h = jnp.dot(x.astype(jnp.bfloat16), w0_ref[...].astype(jnp.bfloat16),
                preferred_element_type=jnp.float32)
    h = h + jnp.dot(g, ws_ref[...].astype(jnp.bfloat16),
                    preferred_element_type=jnp.float32)
    h = jnp.maximum(h, 0.0)
    h = jnp.dot(h.astype(jnp.bfloat16), w1_ref[...].astype(jnp.bfloat16),
                preferred_element_type=jnp.float32)
    h = h + b1_ref[...]
    h = jnp.maximum(h, 0.0)
    o = jnp.sum(h * w2_ref[...], axis=1,
                keepdims=True) + b2_ref[...]
    o_ref[...] = o


def kernel(atom, degree, hybridization, chirality, formal_charge, partial_charge, pos, dist, spatial_distance, bond_type, stereo, conjugated, emb_atom, emb_degree, emb_hybridization, emb_chirality, emb_bond_type, emb_stereo, emb_conjugated, W_charge, b_charge, W_pos, b_pos, Wt_bond_type, bt_bond_type, Wt_stereo, bt_stereo, Wt_conjugated, bt_conjugated, Wf0, bf0, Wf1, bf1, Wf2, bf2):
    b, n = spatial_distance.shape[0], spatial_distance.shape[1]
    d = dist.shape[-1]
    h1 = Wf0.shape[1]
    h2 = Wf1.shape[1]
    r = b * n

    wsmall = pl.pallas_call(
        _prologue,
        out_shape=jax.ShapeDtypeStruct((24, h1), jnp.float32),
    )(emb_bond_type, Wt_bond_type, bt_bond_type.reshape(1, -1),
      emb_stereo, Wt_stereo, bt_stereo.reshape(1, -1),
      emb_conjugated, Wt_conjugated, bt_conjugated.reshape(1, -1),
      Wf0, bf0.reshape(1, -1))

    m = r * n
    tm = min(_TB * n, m)
    lanes = 128
    msub = m // lanes
    nsub = tm // lanes
    dist2 = dist.reshape(m, d)
    sd2 = spatial_distance.reshape(msub, lanes)
    bt2 = bond_type.reshape(msub, lanes)
    st2 = stereo.reshape(msub, lanes)
    cj2 = conjugated.reshape(msub, lanes)

    grid = (m // tm,)
    lane_spec = pl.BlockSpec((nsub, lanes), lambda i: (i, 0))
    col_spec = pl.BlockSpec((tm, 1), lambda i: (i, 0))
    out2 = pl.pallas_call(
        _main,
        grid=grid,
        in_specs=[
            pl.BlockSpec((tm, d), lambda i: (i, 0)),
            lane_spec, lane_spec, lane_spec, lane_spec,
            pl.BlockSpec((d, h1), lambda i: (0, 0)),
            pl.BlockSpec((24, h1), lambda i: (0, 0)),
            pl.BlockSpec((h1, h2), lambda i: (0, 0)),
            pl.BlockSpec((1, h2), lambda i: (0, 0)),
            pl.BlockSpec((1, h2), lambda i: (0, 0)),
            pl.BlockSpec((1, 1), lambda i: (0, 0)),
        ],
        out_specs=col_spec,
        out_shape=jax.ShapeDtypeStruct((m, 1), jnp.float32),
        compiler_params=pltpu.CompilerParams(
            dimension_semantics=("parallel",)),
    )(dist2, sd2, bt2, st2, cj2,
      Wf0[:d], wsmall, Wf1, bf1.reshape(1, -1), Wf2.reshape(1, -1),
      bf2.reshape(1, -1))

    return out2.reshape(b, n, n)
